# baseline (speedup 1.0000x reference)
import functools
import math

import jax
import jax.numpy as jnp
from jax import lax
from jax.experimental import pallas as pl
from jax.experimental.pallas import tpu as pltpu
from jax.experimental.pallas import tpu_sc as plsc

NUM_CORES = 2
NUM_SUBCORES = 16
NW = NUM_CORES * NUM_SUBCORES

MAX_CTX_LEN = 256

CH_ROWS = 1
NBUF = 3
LANES = 16
PE_STREAM_ROWS = 32


def _positional_encoding(seq_len: int, d: int) -> jnp.ndarray:
    full_len = max(seq_len, MAX_CTX_LEN)
    position = jnp.arange(full_len, dtype=jnp.float32)[:, None]
    div_term = jnp.exp(
        jnp.arange(0, d, 2, dtype=jnp.float32) * (-math.log(10000.0) / d)
    )
    pe = jnp.zeros((full_len, d), dtype=jnp.float32)
    pe = pe.at[:, 0::2].set(jnp.sin(position * div_term))
    pe = pe.at[:, 1::2].set(jnp.cos(position * div_term))
    return pe[:seq_len]


@functools.lru_cache(maxsize=None)
def _make_kernel(B: int, S: int, D: int):
    per_w = B * S // NW
    ch = CH_ROWS * S
    nsteps = per_w // ch
    n_pre = PE_STREAM_ROWS
    n_alu = ch - n_pre
    n_main = (nsteps // NBUF) * NBUF
    if n_main >= nsteps:
        n_main -= NBUF
    assert per_w % ch == 0 and n_main >= NBUF
    assert n_pre % 8 == 0 and D % LANES == 0
    mesh = plsc.VectorSubcoreMesh(
        core_axis_name="c",
        subcore_axis_name="s",
        num_cores=NUM_CORES,
        num_subcores=NUM_SUBCORES,
    )

    @functools.partial(
        pl.kernel,
        out_type=jax.ShapeDtypeStruct((B * S, D), jnp.float32),
        mesh=mesh,
        scratch_types=[
            pltpu.VMEM((per_w,), jnp.int32),
            pltpu.VMEM_SHARED((ch, D), jnp.float32),
            pltpu.VMEM((n_alu, D), jnp.float32),
            pltpu.VMEM((NBUF, ch, D), jnp.float32),
        ]
        + [pltpu.SemaphoreType.DMA] * (2 * NBUF + 1),
    )
    def emb_kernel(ids_hbm, table_hbm, pe_hbm, out_hbm, *scratch):
        idx_all = scratch[0]
        pe_sh = scratch[1]
        pe_alu = scratch[2]
        bufs = scratch[3]
        sem_g = scratch[4 : 4 + NBUF]
        sem_wb = scratch[4 + NBUF : 4 + 2 * NBUF]
        sem_p = scratch[4 + 2 * NBUF]
        sid = lax.axis_index("s")
        wid = sid * NUM_CORES + lax.axis_index("c")
        base = wid * per_w

        @pl.when(sid == 0)
        def _load_pe():
            pltpu.sync_copy(pe_hbm, pe_sh)

        pltpu.sync_copy(ids_hbm.at[pl.ds(base, per_w)], idx_all)
        plsc.subcore_barrier()
        pltpu.sync_copy(pe_sh.at[pl.ds(n_pre, n_alu)], pe_alu)

        def prefill(b):
            pltpu.sync_copy(pe_sh.at[pl.ds(0, n_pre)], bufs.at[b].at[pl.ds(0, n_pre)])

        def prefill_async(b):
            pltpu.async_copy(
                pe_sh.at[pl.ds(0, n_pre)], bufs.at[b].at[pl.ds(0, n_pre)], sem_p
            )

        def wait_prefill(b):
            pltpu.make_async_copy(
                pe_sh.at[pl.ds(0, n_pre)], bufs.at[b].at[pl.ds(0, n_pre)], sem_p
            ).wait()

        def out_slice(k):
            return out_hbm.at[pl.ds(base + k * ch, ch)]

        def gather_parts(k, b):
            a = (
                table_hbm.at[idx_all.at[pl.ds(k * ch, n_pre)]],
                bufs.at[b].at[pl.ds(0, n_pre)],
            )
            bb = (
                table_hbm.at[idx_all.at[pl.ds(k * ch + n_pre, n_alu)]],
                bufs.at[b].at[pl.ds(n_pre, n_alu)],
            )
            return a, bb

        def issue_gather(k, b):
            a, bb = gather_parts(k, b)
            pltpu.async_copy(a[0], a[1], sem_g[b], add=True)
            pltpu.async_copy(bb[0], bb[1], sem_g[b])

        def wait_gather(k, b):
            a, bb = gather_parts(k, b)
            pltpu.make_async_copy(a[0], a[1], sem_g[b]).wait()
            pltpu.make_async_copy(bb[0], bb[1], sem_g[b]).wait()

        def alu_add(b):
            buf = bufs.at[b]

            def row_body(i, carry):
                for j in range(D // LANES):
                    sl = pl.ds(j * LANES, LANES)
                    buf[n_pre + i, sl] = buf[n_pre + i, sl] + pe_alu[i, sl]
                return carry

            lax.fori_loop(0, n_alu, row_body, 0)

        def wait_wb(k, b):
            pltpu.make_async_copy(bufs.at[b], out_slice(k), sem_wb[b]).wait()

        def step(k, b, dyn=True):
            b1 = (b + 1) % NBUF
            b2 = (b + 2) % NBUF

            def _launch():
                wait_prefill(b1)
                issue_gather(k + 1, b1)

            def _retire():
                wait_gather(k, b)
                alu_add(b)
                pltpu.async_copy(bufs.at[b], out_slice(k), sem_wb[b])

            def _recycle():
                wait_wb(k - 1, b2)

            def _stage():
                if dyn:
                    pl.when(k >= 1)(_recycle)
                elif k >= 1:
                    _recycle()
                prefill_async(b2)

            if dyn:
                pl.when(k + 1 < nsteps)(_launch)
            elif k + 1 < nsteps:
                _launch()
            _retire()
            if dyn:
                pl.when(k + 2 < nsteps)(_stage)
            elif k + 2 < nsteps:
                _stage()

        prefill(0)
        issue_gather(0, 0)
        prefill_async(1)

        def outer(g, carry):
            for b in range(NBUF):
                step(g * NBUF + b, b)
            return carry

        lax.fori_loop(0, n_main // NBUF, outer, 0)
        for k in range(n_main, nsteps):
            step(k, k % NBUF, dyn=False)

        for k in range(nsteps - NBUF, nsteps):
            wait_wb(k, k % NBUF)

    return emb_kernel


def kernel(token_ids, token_embedding):
    B, S = token_ids.shape
    V, D = token_embedding.shape
    pe = _positional_encoding(S, D)
    pe_rep = jnp.tile(pe, (CH_ROWS, 1))
    ids = token_ids.reshape(B * S).astype(jnp.int32)
    emb_kernel = _make_kernel(B, S, D)
    out = emb_kernel(ids, token_embedding, pe_rep)
    return out.reshape(B, S, D)


# speedup vs baseline: 1.0136x; 1.0136x over previous
import functools
import math

import jax
import jax.numpy as jnp
from jax import lax
from jax.experimental import pallas as pl
from jax.experimental.pallas import tpu as pltpu
from jax.experimental.pallas import tpu_sc as plsc

NUM_CORES = 2
NUM_SUBCORES = 16
NW = NUM_CORES * NUM_SUBCORES

MAX_CTX_LEN = 256

CH_ROWS = 1
NBUF = 4
LANES = 16
PE_STREAM_ROWS = 80


def _positional_encoding(seq_len: int, d: int) -> jnp.ndarray:
    full_len = max(seq_len, MAX_CTX_LEN)
    position = jnp.arange(full_len, dtype=jnp.float32)[:, None]
    div_term = jnp.exp(
        jnp.arange(0, d, 2, dtype=jnp.float32) * (-math.log(10000.0) / d)
    )
    pe = jnp.zeros((full_len, d), dtype=jnp.float32)
    pe = pe.at[:, 0::2].set(jnp.sin(position * div_term))
    pe = pe.at[:, 1::2].set(jnp.cos(position * div_term))
    return pe[:seq_len]


@functools.lru_cache(maxsize=None)
def _make_kernel(B: int, S: int, D: int):
    per_w = B * S // NW
    ch = CH_ROWS * S
    nsteps = per_w // ch
    n_pre = PE_STREAM_ROWS
    n_alu = ch - n_pre
    n_main = (nsteps // NBUF) * NBUF
    if n_main >= nsteps:
        n_main -= NBUF
    assert per_w % ch == 0 and n_main >= NBUF
    assert n_pre % 8 == 0 and D % LANES == 0
    mesh = plsc.VectorSubcoreMesh(
        core_axis_name="c",
        subcore_axis_name="s",
        num_cores=NUM_CORES,
        num_subcores=NUM_SUBCORES,
    )

    @functools.partial(
        pl.kernel,
        out_type=jax.ShapeDtypeStruct((B * S, D), jnp.float32),
        mesh=mesh,
        scratch_types=[
            pltpu.VMEM((per_w,), jnp.int32),
            pltpu.VMEM_SHARED((ch, D), jnp.float32),
            pltpu.VMEM((n_alu, D), jnp.float32),
            pltpu.VMEM((NBUF, ch, D), jnp.float32),
        ]
        + [pltpu.SemaphoreType.DMA] * (2 * NBUF + 1),
    )
    def emb_kernel(ids_hbm, table_hbm, pe_hbm, out_hbm, *scratch):
        idx_all = scratch[0]
        pe_sh = scratch[1]
        pe_alu = scratch[2]
        bufs = scratch[3]
        sem_g = scratch[4 : 4 + NBUF]
        sem_wb = scratch[4 + NBUF : 4 + 2 * NBUF]
        sem_p = scratch[4 + 2 * NBUF]
        sid = lax.axis_index("s")
        wid = sid * NUM_CORES + lax.axis_index("c")
        base = wid * per_w

        @pl.when(sid == 0)
        def _load_pe():
            pltpu.sync_copy(pe_hbm, pe_sh)

        pltpu.sync_copy(ids_hbm.at[pl.ds(base, per_w)], idx_all)
        plsc.subcore_barrier()
        pltpu.sync_copy(pe_sh.at[pl.ds(n_pre, n_alu)], pe_alu)

        def prefill(b):
            pltpu.sync_copy(pe_sh.at[pl.ds(0, n_pre)], bufs.at[b].at[pl.ds(0, n_pre)])

        def prefill_async(b):
            pltpu.async_copy(
                pe_sh.at[pl.ds(0, n_pre)], bufs.at[b].at[pl.ds(0, n_pre)], sem_p
            )

        def wait_prefill(b):
            pltpu.make_async_copy(
                pe_sh.at[pl.ds(0, n_pre)], bufs.at[b].at[pl.ds(0, n_pre)], sem_p
            ).wait()

        def out_slice(k):
            return out_hbm.at[pl.ds(base + k * ch, ch)]

        def gather_parts(k, b):
            a = (
                table_hbm.at[idx_all.at[pl.ds(k * ch, n_pre)]],
                bufs.at[b].at[pl.ds(0, n_pre)],
            )
            bb = (
                table_hbm.at[idx_all.at[pl.ds(k * ch + n_pre, n_alu)]],
                bufs.at[b].at[pl.ds(n_pre, n_alu)],
            )
            return a, bb

        def issue_gather(k, b):
            a, bb = gather_parts(k, b)
            pltpu.async_copy(a[0], a[1], sem_g[b], add=True)
            pltpu.async_copy(bb[0], bb[1], sem_g[b])

        def wait_gather(k, b):
            a, bb = gather_parts(k, b)
            pltpu.make_async_copy(a[0], a[1], sem_g[b]).wait()
            pltpu.make_async_copy(bb[0], bb[1], sem_g[b]).wait()

        def alu_add(b):
            buf = bufs.at[b]

            def row_body(i, carry):
                for j in range(D // LANES):
                    sl = pl.ds(j * LANES, LANES)
                    buf[n_pre + i, sl] = buf[n_pre + i, sl] + pe_alu[i, sl]
                return carry

            lax.fori_loop(0, n_alu, row_body, 0)

        def wait_wb(k, b):
            pltpu.make_async_copy(bufs.at[b], out_slice(k), sem_wb[b]).wait()

        def step(k, b, dyn=True):
            b1 = (b + 1) % NBUF
            b2 = (b + 2) % NBUF

            def _launch():
                wait_prefill(b1)
                issue_gather(k + 1, b1)

            def _retire():
                wait_gather(k, b)
                alu_add(b)
                pltpu.async_copy(bufs.at[b], out_slice(k), sem_wb[b])

            def _recycle():
                wait_wb(k + 2 - NBUF, b2)

            def _stage():
                if dyn:
                    pl.when(k >= NBUF - 2)(_recycle)
                elif k >= NBUF - 2:
                    _recycle()
                prefill_async(b2)

            if dyn:
                pl.when(k + 1 < nsteps)(_launch)
            elif k + 1 < nsteps:
                _launch()
            _retire()
            if dyn:
                pl.when(k + 2 < nsteps)(_stage)
            elif k + 2 < nsteps:
                _stage()

        prefill(0)
        issue_gather(0, 0)
        prefill_async(1)

        def outer(g, carry):
            for b in range(NBUF):
                step(g * NBUF + b, b)
            return carry

        lax.fori_loop(0, n_main // NBUF, outer, 0)
        for k in range(n_main, nsteps):
            step(k, k % NBUF, dyn=False)

        for k in range(nsteps - NBUF, nsteps):
            wait_wb(k, k % NBUF)

    return emb_kernel


def kernel(token_ids, token_embedding):
    B, S = token_ids.shape
    V, D = token_embedding.shape
    pe = _positional_encoding(S, D)
    pe_rep = jnp.tile(pe, (CH_ROWS, 1))
    ids = token_ids.reshape(B * S).astype(jnp.int32)
    emb_kernel = _make_kernel(B, S, D)
    out = emb_kernel(ids, token_embedding, pe_rep)
    return out.reshape(B, S, D)


# speedup vs baseline: 1.0400x; 1.0260x over previous
import functools
import math

import jax
import jax.numpy as jnp
from jax import lax
from jax.experimental import pallas as pl
from jax.experimental.pallas import tpu as pltpu
from jax.experimental.pallas import tpu_sc as plsc

NUM_CORES = 2
NUM_SUBCORES = 16
NW = NUM_CORES * NUM_SUBCORES

MAX_CTX_LEN = 256

CH_ROWS = 1
NBUF = 4
GSPLIT = 2


def _positional_encoding(seq_len: int, d: int) -> jnp.ndarray:
    full_len = max(seq_len, MAX_CTX_LEN)
    position = jnp.arange(full_len, dtype=jnp.float32)[:, None]
    div_term = jnp.exp(
        jnp.arange(0, d, 2, dtype=jnp.float32) * (-math.log(10000.0) / d)
    )
    pe = jnp.zeros((full_len, d), dtype=jnp.float32)
    pe = pe.at[:, 0::2].set(jnp.sin(position * div_term))
    pe = pe.at[:, 1::2].set(jnp.cos(position * div_term))
    return pe[:seq_len]


@functools.lru_cache(maxsize=None)
def _make_kernel(B: int, S: int, D: int):
    per_w = B * S // NW
    ch = CH_ROWS * S
    nsteps = per_w // ch
    cuts = [8 * round(ch * j / GSPLIT / 8) for j in range(GSPLIT + 1)]
    parts = [(cuts[j], cuts[j + 1] - cuts[j]) for j in range(GSPLIT)]
    n_main = (nsteps // NBUF) * NBUF
    if n_main >= nsteps:
        n_main -= NBUF
    assert per_w % ch == 0 and all(n > 0 for _, n in parts)
    assert n_main >= NBUF
    mesh = plsc.VectorSubcoreMesh(
        core_axis_name="c",
        subcore_axis_name="s",
        num_cores=NUM_CORES,
        num_subcores=NUM_SUBCORES,
    )

    @functools.partial(
        pl.kernel,
        out_type=jax.ShapeDtypeStruct((B * S, D), jnp.float32),
        mesh=mesh,
        scratch_types=[
            pltpu.VMEM((per_w,), jnp.int32),
            pltpu.VMEM_SHARED((ch, D), jnp.float32),
            pltpu.VMEM((NBUF, ch, D), jnp.float32),
        ]
        + [pltpu.SemaphoreType.DMA] * (2 * NBUF + 1),
    )
    def emb_kernel(ids_hbm, table_hbm, pe_hbm, out_hbm, *scratch):
        idx_all = scratch[0]
        pe_sh = scratch[1]
        bufs = scratch[2]
        sem_g = scratch[3 : 3 + NBUF]
        sem_wb = scratch[3 + NBUF : 3 + 2 * NBUF]
        sem_p = scratch[3 + 2 * NBUF]
        sid = lax.axis_index("s")
        wid = sid * NUM_CORES + lax.axis_index("c")
        base = wid * per_w

        @pl.when(sid == 0)
        def _load_pe():
            pltpu.sync_copy(pe_hbm, pe_sh)

        pltpu.sync_copy(ids_hbm.at[pl.ds(base, per_w)], idx_all)
        plsc.subcore_barrier()

        def prefill(b):
            pltpu.sync_copy(pe_sh, bufs.at[b])

        def prefill_async(b):
            pltpu.async_copy(pe_sh, bufs.at[b], sem_p)

        def wait_prefill(b):
            pltpu.make_async_copy(pe_sh, bufs.at[b], sem_p).wait()

        def out_slice(k):
            return out_hbm.at[pl.ds(base + k * ch, ch)]

        def gather_parts(k, b):
            return [
                (
                    table_hbm.at[idx_all.at[pl.ds(k * ch + off, n)]],
                    bufs.at[b].at[pl.ds(off, n)],
                )
                for off, n in parts
            ]

        def issue_gather(k, b):
            for src, dst in gather_parts(k, b):
                pltpu.async_copy(src, dst, sem_g[b], add=True)

        def wait_gather(k, b):
            for src, dst in gather_parts(k, b):
                pltpu.make_async_copy(src, dst, sem_g[b]).wait()

        def wait_wb(k, b):
            pltpu.make_async_copy(bufs.at[b], out_slice(k), sem_wb[b]).wait()

        def step(k, b, dyn=True):
            b1 = (b + 1) % NBUF
            b2 = (b + 2) % NBUF

            def _launch():
                wait_prefill(b1)
                issue_gather(k + 1, b1)

            def _retire():
                wait_gather(k, b)
                pltpu.async_copy(bufs.at[b], out_slice(k), sem_wb[b])

            def _recycle():
                wait_wb(k + 2 - NBUF, b2)

            def _stage():
                if dyn:
                    pl.when(k >= NBUF - 2)(_recycle)
                elif k >= NBUF - 2:
                    _recycle()
                prefill_async(b2)

            if dyn:
                pl.when(k + 1 < nsteps)(_launch)
            elif k + 1 < nsteps:
                _launch()
            if dyn:
                pl.when(k + 2 < nsteps)(_stage)
            elif k + 2 < nsteps:
                _stage()
            _retire()

        prefill(0)
        issue_gather(0, 0)
        prefill_async(1)

        def outer(g, carry):
            for b in range(NBUF):
                step(g * NBUF + b, b)
            return carry

        lax.fori_loop(0, n_main // NBUF, outer, 0)
        for k in range(n_main, nsteps):
            step(k, k % NBUF, dyn=False)

        for k in range(nsteps - NBUF, nsteps):
            wait_wb(k, k % NBUF)

    return emb_kernel


def kernel(token_ids, token_embedding):
    B, S = token_ids.shape
    V, D = token_embedding.shape
    pe = _positional_encoding(S, D)
    pe_rep = jnp.tile(pe, (CH_ROWS, 1))
    ids = token_ids.reshape(B * S).astype(jnp.int32)
    emb_kernel = _make_kernel(B, S, D)
    out = emb_kernel(ids, token_embedding, pe_rep)
    return out.reshape(B, S, D)


# speedup vs baseline: 1.0478x; 1.0075x over previous
import functools
import math

import jax
import jax.numpy as jnp
from jax import lax
from jax.experimental import pallas as pl
from jax.experimental.pallas import tpu as pltpu
from jax.experimental.pallas import tpu_sc as plsc

NUM_CORES = 2
NUM_SUBCORES = 16
NW = NUM_CORES * NUM_SUBCORES

MAX_CTX_LEN = 256

CH_ROWS = 1
NBUF = 4
GSPLIT = 1


def _positional_encoding(seq_len: int, d: int) -> jnp.ndarray:
    full_len = max(seq_len, MAX_CTX_LEN)
    position = jnp.arange(full_len, dtype=jnp.float32)[:, None]
    div_term = jnp.exp(
        jnp.arange(0, d, 2, dtype=jnp.float32) * (-math.log(10000.0) / d)
    )
    pe = jnp.zeros((full_len, d), dtype=jnp.float32)
    pe = pe.at[:, 0::2].set(jnp.sin(position * div_term))
    pe = pe.at[:, 1::2].set(jnp.cos(position * div_term))
    return pe[:seq_len]


@functools.lru_cache(maxsize=None)
def _make_kernel(B: int, S: int, D: int):
    per_w = B * S // NW
    ch = CH_ROWS * S
    nsteps = per_w // ch
    cuts = [8 * round(ch * j / GSPLIT / 8) for j in range(GSPLIT + 1)]
    parts = [(cuts[j], cuts[j + 1] - cuts[j]) for j in range(GSPLIT)]
    n_main = (nsteps // NBUF) * NBUF
    if n_main >= nsteps:
        n_main -= NBUF
    assert per_w % ch == 0 and all(n > 0 for _, n in parts)
    assert n_main >= NBUF
    mesh = plsc.VectorSubcoreMesh(
        core_axis_name="c",
        subcore_axis_name="s",
        num_cores=NUM_CORES,
        num_subcores=NUM_SUBCORES,
    )

    @functools.partial(
        pl.kernel,
        out_type=jax.ShapeDtypeStruct((B * S, D), jnp.float32),
        mesh=mesh,
        scratch_types=[
            pltpu.VMEM((per_w,), jnp.int32),
            pltpu.VMEM_SHARED((ch, D), jnp.float32),
            pltpu.VMEM((NBUF, ch, D), jnp.float32),
        ]
        + [pltpu.SemaphoreType.DMA] * (2 * NBUF + 1),
    )
    def emb_kernel(ids_hbm, table_hbm, pe_hbm, out_hbm, *scratch):
        idx_all = scratch[0]
        pe_sh = scratch[1]
        bufs = scratch[2]
        sem_g = scratch[3 : 3 + NBUF]
        sem_wb = scratch[3 + NBUF : 3 + 2 * NBUF]
        sem_p = scratch[3 + 2 * NBUF]
        sid = lax.axis_index("s")
        wid = sid * NUM_CORES + lax.axis_index("c")
        base = wid * per_w

        idx_load = pltpu.async_copy(
            ids_hbm.at[pl.ds(base, per_w)], idx_all, sem_p
        )

        @pl.when(sid == 0)
        def _load_pe():
            pltpu.sync_copy(pe_hbm, pe_sh)

        plsc.subcore_barrier()
        idx_load.wait()

        def prefill(b):
            pltpu.sync_copy(pe_sh, bufs.at[b])

        def prefill_async(b):
            pltpu.async_copy(pe_sh, bufs.at[b], sem_p)

        def wait_prefill(b):
            pltpu.make_async_copy(pe_sh, bufs.at[b], sem_p).wait()

        def out_slice(k):
            return out_hbm.at[pl.ds(base + k * ch, ch)]

        def gather_parts(k, b):
            return [
                (
                    table_hbm.at[idx_all.at[pl.ds(k * ch + off, n)]],
                    bufs.at[b].at[pl.ds(off, n)],
                )
                for off, n in parts
            ]

        def issue_gather(k, b):
            for src, dst in gather_parts(k, b):
                pltpu.async_copy(src, dst, sem_g[b], add=True)

        def wait_gather(k, b):
            for src, dst in gather_parts(k, b):
                pltpu.make_async_copy(src, dst, sem_g[b]).wait()

        def wait_wb(k, b):
            pltpu.make_async_copy(bufs.at[b], out_slice(k), sem_wb[b]).wait()

        def step(k, b, dyn=True):
            b1 = (b + 1) % NBUF
            b2 = (b + 2) % NBUF

            def _launch():
                wait_prefill(b1)
                issue_gather(k + 1, b1)

            def _retire():
                wait_gather(k, b)
                pltpu.async_copy(bufs.at[b], out_slice(k), sem_wb[b])

            def _recycle():
                wait_wb(k + 2 - NBUF, b2)

            def _stage():
                if dyn:
                    pl.when(k >= NBUF - 2)(_recycle)
                elif k >= NBUF - 2:
                    _recycle()
                prefill_async(b2)

            if dyn:
                pl.when(k + 1 < nsteps)(_launch)
            elif k + 1 < nsteps:
                _launch()
            if dyn:
                pl.when(k + 2 < nsteps)(_stage)
            elif k + 2 < nsteps:
                _stage()
            _retire()

        prefill(0)
        issue_gather(0, 0)
        prefill_async(1)

        def outer(g, carry):
            for b in range(NBUF):
                step(g * NBUF + b, b)
            return carry

        lax.fori_loop(0, n_main // NBUF, outer, 0)
        for k in range(n_main, nsteps):
            step(k, k % NBUF, dyn=False)

        for k in range(nsteps - NBUF, nsteps):
            wait_wb(k, k % NBUF)

    return emb_kernel


def kernel(token_ids, token_embedding):
    B, S = token_ids.shape
    V, D = token_embedding.shape
    pe = _positional_encoding(S, D)
    pe_rep = jnp.tile(pe, (CH_ROWS, 1))
    ids = token_ids.reshape(B * S).astype(jnp.int32)
    emb_kernel = _make_kernel(B, S, D)
    out = emb_kernel(ids, token_embedding, pe_rep)
    return out.reshape(B, S, D)
